# speedup vs baseline: 10.3382x; 10.3382x over previous
import functools

import jax
import jax.numpy as jnp
from jax import lax
from jax.experimental import pallas as pl
from jax.experimental.pallas import tpu as pltpu
from jax.experimental.pallas import tpu_sc as plsc

IN_F = 256
OUT_F = 256
NUM_E = 64
BATCH = 2048

TILE = 256
NUM_TILES = BATCH // TILE
G = NUM_TILES + NUM_E - 1

SC_CORES = 2
SC_SUBCORES = 16
SC_WORKERS = SC_CORES * SC_SUBCORES


def _sc_row_gather(table, idx):
    n = idx.shape[0]
    d = table.shape[1]
    rows_per_w = n // SC_WORKERS
    mesh = plsc.VectorSubcoreMesh(core_axis_name="c", subcore_axis_name="s")

    @functools.partial(
        pl.kernel,
        out_type=jax.ShapeDtypeStruct((n, d), table.dtype),
        mesh=mesh,
        scratch_types=[
            pltpu.VMEM((rows_per_w,), jnp.int32),
            pltpu.VMEM((rows_per_w, d), table.dtype),
            pltpu.SemaphoreType.DMA,
        ],
    )
    def k(table_hbm, idx_hbm, out_hbm, idx_v, rows_v, sem):
        wid = lax.axis_index("s") * SC_CORES + lax.axis_index("c")
        base = wid * rows_per_w
        pltpu.sync_copy(idx_hbm.at[pl.ds(base, rows_per_w)], idx_v)
        pltpu.async_copy(table_hbm.at[idx_v], rows_v, sem).wait()
        pltpu.sync_copy(rows_v, out_hbm.at[pl.ds(base, rows_per_w)])

    return k(table, idx)


def _gmm_body(tile_r, e_r, lo_r, hi_r, x_ref, w_ref, wf_ref, b_ref, bf_ref, o_ref):
    g = pl.program_id(0)
    t = tile_r[g]
    lo = lo_r[g]
    hi = hi_r[g]
    rows = t * TILE + lax.broadcasted_iota(jnp.int32, (TILE, 1), 0)
    mask = (rows >= lo) & (rows < hi)
    x = jnp.where(mask, x_ref[...], 0.0).astype(jnp.bfloat16)
    w = (w_ref[0] + wf_ref[...]).astype(jnp.bfloat16)
    acc = lax.dot_general(
        x, w, (((1,), (1,)), ((), ())), preferred_element_type=jnp.float32
    )
    brow = b_ref[0] + bf_ref[...]
    acc = acc + jnp.where(mask, brow, 0.0)
    first = jnp.logical_or(g == 0, t != tile_r[jnp.maximum(g - 1, 0)])

    @pl.when(first)
    def _():
        o_ref[...] = acc

    @pl.when(jnp.logical_not(first))
    def _():
        o_ref[...] += acc


def _grouped_matmul(tile_of, e_of, lo, hi, x_sorted, w3, wf2, bias3, bf2):
    grid_spec = pltpu.PrefetchScalarGridSpec(
        num_scalar_prefetch=4,
        grid=(G,),
        in_specs=[
            pl.BlockSpec((TILE, IN_F), lambda g, tr, er, lr, hr: (tr[g], 0)),
            pl.BlockSpec((1, OUT_F, IN_F), lambda g, tr, er, lr, hr: (er[g], 0, 0)),
            pl.BlockSpec((OUT_F, IN_F), lambda g, tr, er, lr, hr: (0, 0)),
            pl.BlockSpec((1, 1, OUT_F), lambda g, tr, er, lr, hr: (er[g], 0, 0)),
            pl.BlockSpec((1, OUT_F), lambda g, tr, er, lr, hr: (0, 0)),
        ],
        out_specs=pl.BlockSpec((TILE, OUT_F), lambda g, tr, er, lr, hr: (tr[g], 0)),
    )
    return pl.pallas_call(
        _gmm_body,
        grid_spec=grid_spec,
        out_shape=jax.ShapeDtypeStruct((BATCH, OUT_F), jnp.float32),
        compiler_params=pltpu.CompilerParams(dimension_semantics=("arbitrary",)),
    )(tile_of, e_of, lo, hi, x_sorted, w3, wf2, bias3, bf2)


def kernel(input, route_index, weight, weight_fact, bias, bias_fact):
    r = route_index.astype(jnp.int32)
    perm = jnp.argsort(r).astype(jnp.int32)
    inv = jnp.zeros((BATCH,), jnp.int32).at[perm].set(
        jnp.arange(BATCH, dtype=jnp.int32)
    )

    counts = jnp.bincount(r, length=NUM_E)
    off = jnp.concatenate(
        [jnp.zeros((1,), jnp.int32), jnp.cumsum(counts).astype(jnp.int32)]
    )
    first_t = off[:NUM_E] // TILE
    nonempty = counts > 0
    last_t = jnp.where(nonempty, (off[1:] - 1) // TILE, 0)
    ntiles = jnp.where(nonempty, last_t - first_t + 1, 0)
    cum = jnp.cumsum(ntiles)
    starts = cum - ntiles
    gids = jnp.arange(G)
    e_g = jnp.searchsorted(cum, gids, side="right")
    valid = e_g < NUM_E
    e_safe = jnp.where(valid, e_g, 0).astype(jnp.int32)
    t_g = first_t[e_safe] + (gids - starts[e_safe])
    tile_of = jnp.where(valid, t_g, NUM_TILES - 1).astype(jnp.int32)
    lo = jnp.where(valid, off[e_safe], 0).astype(jnp.int32)
    hi = jnp.where(valid, off[e_safe + 1], 0).astype(jnp.int32)

    x_sorted = _sc_row_gather(input, perm)
    w3 = weight.reshape(NUM_E, OUT_F, IN_F)
    wf2 = weight_fact.reshape(OUT_F, IN_F)
    bias3 = bias.reshape(NUM_E, 1, OUT_F)
    y_sorted = _grouped_matmul(
        tile_of, e_safe, lo, hi, x_sorted, w3, wf2, bias3, bias_fact
    )
    return _sc_row_gather(y_sorted, inv)
